# baseline (speedup 1.0000x reference)
import functools

import jax
import jax.numpy as jnp
from jax import lax
from jax.experimental import pallas as pl
from jax.experimental.pallas import tpu as pltpu
from jax.experimental.pallas import tpu_sc as plsc

N = 10000
E = 160000
D = 128
H = 16
C = 64
HD = H * C

NC = 2
NS = 16
NW = NC * NS
EW = 5008
E_PAD = NW * EW
CHUNKS = EW // 16
NPT = N // NS
ET = E // NS
HSC = H // NC

_f32 = jnp.float32
_i32 = jnp.int32


def _mm_body(x_ref, wl_ref, wr_ref, xl_ref, xr_ref):
    xb = x_ref[...]
    xl_ref[...] = jnp.dot(xb, wl_ref[...], preferred_element_type=_f32,
                          precision=lax.Precision.HIGHEST)
    xr_ref[...] = jnp.dot(xb, wr_ref[...], preferred_element_type=_f32,
                          precision=lax.Precision.HIGHEST)


def _k1_xlxr(x, W_l, W_r):
    return pl.pallas_call(
        _mm_body,
        grid=(10,),
        in_specs=[
            pl.BlockSpec((N // 10, D), lambda i: (i, 0)),
            pl.BlockSpec((D, HD), lambda i: (0, 0)),
            pl.BlockSpec((D, HD), lambda i: (0, 0)),
        ],
        out_specs=[
            pl.BlockSpec((N // 10, HD), lambda i: (i, 0)),
            pl.BlockSpec((N // 10, HD), lambda i: (i, 0)),
        ],
        out_shape=[
            jax.ShapeDtypeStruct((N, HD), _f32),
            jax.ShapeDtypeStruct((N, HD), _f32),
        ],
    )(x, W_l, W_r)


def _passa_body(xl_hbm, xr_hbm, src_hbm, dst_hbm, att_hbm, z16_hbm,
                p_hbm, dparts_hbm,
                sidx, didx, attv, gxl, gxr, pstage, dbuf, dnsp):
    ci = lax.axis_index("c")
    si = lax.axis_index("s")
    wid = ci * NS + si
    base = wid * EW
    nvalid = (jnp.minimum(E, base + EW) - base) // 16
    pltpu.sync_copy(src_hbm.at[pl.ds(base, EW)], sidx)
    pltpu.sync_copy(dst_hbm.at[pl.ds(base, EW)], didx)
    pltpu.sync_copy(att_hbm, attv)
    pltpu.sync_copy(z16_hbm.at[si], dbuf)
    pltpu.sync_copy(dbuf, dnsp.at[pl.ds(si * NPT, NPT)])
    plsc.subcore_barrier()
    iot = lax.iota(_i32, 16)

    def chunk(i, carry):
        sv = sidx[pl.ds(i * 16, 16)]
        dv = didx[pl.ds(i * 16, 16)]
        pltpu.sync_copy(xl_hbm.at[sv], gxl)
        pltpu.sync_copy(xr_hbm.at[dv], gxr)

        def hloop(h, hcarry):
            acc = jnp.zeros((16,), _f32)
            for grp in range(C // 16):
                av = attv[pl.ds(h * C + grp * 16, 16)]
                for cc in range(16):
                    j = h * C + grp * 16 + cc
                    colj = jnp.full((16,), j, _i32)
                    a = plsc.load_gather(gxl, [iot, colj])
                    b = plsc.load_gather(gxr, [iot, colj])
                    s = a + b
                    lr = jnp.maximum(s, 0.0) + 0.2 * jnp.minimum(s, 0.0)
                    acc = acc + lr * av[cc]
            ph = jnp.exp(acc)
            plsc.store_scatter(pstage, [iot, jnp.full((16,), h, _i32)], ph)
            return hcarry

        lax.fori_loop(0, H, hloop, 0)
        pltpu.sync_copy(pstage, p_hbm.at[pl.ds(base + i * 16, 16)])
        pltpu.sync_copy(pstage, dnsp.at[dv], add=True)
        return carry

    lax.fori_loop(0, nvalid, chunk, 0)
    plsc.subcore_barrier()
    pltpu.sync_copy(dnsp.at[pl.ds(si * NPT, NPT)], dbuf)
    pltpu.sync_copy(dbuf, dparts_hbm.at[ci, si])


def _k2_passa(XL, XR, src_pad, dst_pad, att_flat, zeros16):
    mesh = plsc.VectorSubcoreMesh(core_axis_name="c", subcore_axis_name="s")
    return pl.kernel(
        _passa_body,
        compiler_params=pltpu.CompilerParams(use_tc_tiling_on_sc=False, needs_layout_passes=False),
        out_type=[
            jax.ShapeDtypeStruct((E, H), _f32),
            jax.ShapeDtypeStruct((NC, NS, NPT, H), _f32),
        ],
        mesh=mesh,
        scratch_types=[
            pltpu.VMEM((EW,), _i32),
            pltpu.VMEM((EW,), _i32),
            pltpu.VMEM((HD,), _f32),
            pltpu.VMEM((16, HD), _f32),
            pltpu.VMEM((16, HD), _f32),
            pltpu.VMEM((16, H), _f32),
            pltpu.VMEM((NPT, H), _f32),
            pltpu.VMEM_SHARED((N, H), _f32),
        ],
    )(XL, XR, src_pad, dst_pad, att_flat, zeros16)


def _merge_body(d_ref, out_ref):
    out_ref[...] = (d_ref[0] + d_ref[1]).reshape(N, H)


def _k3_merge(dparts):
    return pl.pallas_call(
        _merge_body,
        out_shape=jax.ShapeDtypeStruct((N, H), _f32),
    )(dparts)


def _passb_body(xl2_hbm, p_hbm, dn_hbm, src_hbm, dst_hbm, t_hbm, z64_hbm,
                s2o_hbm, acco_hbm,
                sidx, didx, tv, gbuf, pbuf, dbuf, ezst, emst,
                s2sp, accsp):
    ci = lax.axis_index("c")
    si = lax.axis_index("s")
    ebase = si * ET
    pltpu.sync_copy(src_hbm.at[pl.ds(ebase, ET)], sidx)
    pltpu.sync_copy(dst_hbm.at[pl.ds(ebase, ET)], didx)
    pltpu.sync_copy(t_hbm, tv)
    iot = lax.iota(_i32, 16)
    tvec = tv[...]

    def head(h, carry):
        hh = ci * HSC + h
        pltpu.sync_copy(z64_hbm.at[si], s2sp.at[pl.ds(si * NPT, NPT)])
        pltpu.sync_copy(z64_hbm.at[si], accsp.at[pl.ds(si * NPT, NPT)])
        plsc.subcore_barrier()
        colh = jnp.full((16,), hh, _i32)

        def chunk(i, icarry):
            sv = sidx[pl.ds(i * 16, 16)]
            dv = didx[pl.ds(i * 16, 16)]
            pltpu.sync_copy(xl2_hbm.at[sv * H + hh], gbuf)
            pltpu.sync_copy(p_hbm.at[pl.ds(ebase + i * 16, 16)], pbuf)
            pltpu.sync_copy(dn_hbm.at[dv], dbuf)
            pv = plsc.load_gather(pbuf, [iot, colh])
            dnv = plsc.load_gather(dbuf, [iot, colh])
            alpha = pv / (dnv + 1e-16)
            ta = alpha * tvec
            for cc in range(C):
                colc = jnp.full((16,), cc, _i32)
                xlv = plsc.load_gather(gbuf, [iot, colc])
                m = alpha * xlv
                ez = jnp.exp(ta * xlv)
                plsc.store_scatter(ezst, [iot, colc], ez)
                plsc.store_scatter(emst, [iot, colc], ez * m)
            pltpu.sync_copy(ezst, s2sp.at[dv], add=True)
            pltpu.sync_copy(emst, accsp.at[dv], add=True)
            return icarry

        lax.fori_loop(0, ET // 16, chunk, 0)
        plsc.subcore_barrier()
        pltpu.sync_copy(s2sp.at[pl.ds(si * NPT, NPT)], s2o_hbm.at[hh, si])
        pltpu.sync_copy(accsp.at[pl.ds(si * NPT, NPT)], acco_hbm.at[hh, si])
        plsc.subcore_barrier()
        return carry

    lax.fori_loop(0, HSC, head, 0)


def _k4_passb(XL2, p, denom, src_pad, dst_pad, t16, zeros64):
    mesh = plsc.VectorSubcoreMesh(core_axis_name="c", subcore_axis_name="s")
    return pl.kernel(
        _passb_body,
        compiler_params=pltpu.CompilerParams(use_tc_tiling_on_sc=False, needs_layout_passes=False),
        out_type=[
            jax.ShapeDtypeStruct((H, NS, NPT, C), _f32),
            jax.ShapeDtypeStruct((H, NS, NPT, C), _f32),
        ],
        mesh=mesh,
        scratch_types=[
            pltpu.VMEM((ET,), _i32),
            pltpu.VMEM((ET,), _i32),
            pltpu.VMEM((16,), _f32),
            pltpu.VMEM((16, C), _f32),
            pltpu.VMEM((16, H), _f32),
            pltpu.VMEM((16, H), _f32),
            pltpu.VMEM((16, C), _f32),
            pltpu.VMEM((16, C), _f32),
            pltpu.VMEM_SHARED((N, C), _f32),
            pltpu.VMEM_SHARED((N, C), _f32),
        ],
    )(XL2, p, denom, src_pad, dst_pad, t16, zeros64)


_NB5 = 1000


def _k5_body(s2_ref, acc_ref, bias_ref, w_ref, g_ref):
    h1 = jnp.maximum(
        acc_ref[...] / (s2_ref[...] + 1e-16) + bias_ref[...][None, :], 0.0)
    g_ref[...] = jnp.dot(h1, w_ref[...], preferred_element_type=_f32,
                         precision=lax.Precision.HIGHEST)


def _k5_g(S2n, Accn, bias1, Wcat):
    return pl.pallas_call(
        _k5_body,
        grid=(N // _NB5,),
        in_specs=[
            pl.BlockSpec((_NB5, HD), lambda r: (r, 0)),
            pl.BlockSpec((_NB5, HD), lambda r: (r, 0)),
            pl.BlockSpec((HD,), lambda r: (0,)),
            pl.BlockSpec((HD, 3), lambda r: (0, 0)),
        ],
        out_specs=pl.BlockSpec((_NB5, 3), lambda r: (r, 0)),
        out_shape=jax.ShapeDtypeStruct((N, 3), _f32),
    )(S2n, Accn, bias1, Wcat)


def _passc_body(gn_hbm, src_hbm, dst_hbm, zn_hbm, nbrp_hbm,
                sidx, didx, gnv, nbuf):
    ci = lax.axis_index("c")
    si = lax.axis_index("s")
    wid = ci * NS + si
    base = wid * EW
    nvalid = (jnp.minimum(E, base + EW) - base) // 16
    pltpu.sync_copy(src_hbm.at[pl.ds(base, EW)], sidx)
    pltpu.sync_copy(dst_hbm.at[pl.ds(base, EW)], didx)
    pltpu.sync_copy(gn_hbm, gnv)
    pltpu.sync_copy(zn_hbm, nbuf)

    def chunk(i, carry):
        sv = sidx[pl.ds(i * 16, 16)]
        dv = didx[pl.ds(i * 16, 16)]
        v = plsc.load_gather(gnv, [sv])
        plsc.addupdate_scatter(nbuf, [dv], v)
        return carry

    lax.fori_loop(0, nvalid, chunk, 0)
    pltpu.sync_copy(nbuf, nbrp_hbm.at[wid])


def _k6_passc(gn, src_pad, dst_pad, zerosN):
    mesh = plsc.VectorSubcoreMesh(core_axis_name="c", subcore_axis_name="s")
    return pl.kernel(
        _passc_body,
        compiler_params=pltpu.CompilerParams(use_tc_tiling_on_sc=False, needs_layout_passes=False),
        out_type=jax.ShapeDtypeStruct((NW, N), _f32),
        mesh=mesh,
        scratch_types=[
            pltpu.VMEM((EW,), _i32),
            pltpu.VMEM((EW,), _i32),
            pltpu.VMEM((N,), _f32),
            pltpu.VMEM((N,), _f32),
        ],
    )(gn, src_pad, dst_pad, zerosN)


_SORT_N = 16384
_SR = 128
_SC_ = 128


def _xor_partner(A, j):
    if j < _SC_:
        right = jnp.concatenate([A[:, _SC_ - j:], A[:, :_SC_ - j]], axis=1)
        left = jnp.concatenate([A[:, j:], A[:, :j]], axis=1)
        m = (lax.broadcasted_iota(_i32, (_SR, _SC_), 1) & j) != 0
    else:
        jj = j // _SC_
        right = jnp.concatenate([A[_SR - jj:, :], A[:_SR - jj, :]], axis=0)
        left = jnp.concatenate([A[jj:, :], A[:jj, :]], axis=0)
        m = (lax.broadcasted_iota(_i32, (_SR, _SC_), 0) & jj) != 0
    return jnp.where(m, right, left)


def _k7_body(g_ref, nbrp_ref, bg_ref, bout_ref, o_ref):
    nbr = jnp.sum(nbrp_ref[...], axis=0)
    gmat = g_ref[...]
    gr = gmat[:, 0]
    q = gmat[:, 2]
    score = jnp.tanh(gr + nbr + bg_ref[0])
    sp = jnp.concatenate([score, jnp.full((_SORT_N - N,), -2.0, _f32)])
    qp = jnp.concatenate([q, jnp.zeros((_SORT_N - N,), _f32)])
    S = sp.reshape(_SR, _SC_)
    Q = qp.reshape(_SR, _SC_)
    pos = (lax.broadcasted_iota(_i32, (_SR, _SC_), 0) * _SC_
           + lax.broadcasted_iota(_i32, (_SR, _SC_), 1))
    I = pos

    k = 2
    while k <= _SORT_N:
        j = k // 2
        while j >= 1:
            Sp = _xor_partner(S, j)
            Ip = _xor_partner(I, j)
            Qp = _xor_partner(Q, j)
            bitj = (pos & j) != 0
            dirm = (pos & k) == 0
            want_larger = jnp.logical_xor(dirm, bitj)
            a_before = (S > Sp) | ((S == Sp) & (I < Ip))
            take_a = want_larger == a_before
            S = jnp.where(take_a, S, Sp)
            I = jnp.where(take_a, I, Ip)
            Q = jnp.where(take_a, Q, Qp)
            j //= 2
        k *= 2

    o_ref[...] = jnp.maximum(S, 0.0) * Q + bout_ref[0]


def _k7_out(G, nbrp, bg, bout):
    return pl.pallas_call(
        _k7_body,
        out_shape=jax.ShapeDtypeStruct((_SR, _SC_), _f32),
    )(G, nbrp, bg, bout)


def kernel(x, edge_attr, W_l, W_r, att, bias1, t, Wg_root, Wg_nbr, bg, Wout,
           bout, edge_index):
    src_pad = jnp.pad(edge_index[0], (0, E_PAD - E))
    dst_pad = jnp.pad(edge_index[1], (0, E_PAD - E))
    att_flat = att.reshape(HD)
    zeros16 = jnp.zeros((NS, NPT, H), _f32)
    zeros64 = jnp.zeros((NS, NPT, C), _f32)
    zerosN = jnp.zeros((N,), _f32)
    t16 = jnp.broadcast_to(t, (16,))
    Wcat = jnp.concatenate([Wg_root, Wg_nbr, Wout], axis=1)

    XL, XR = _k1_xlxr(x, W_l, W_r)
    p, dparts = _k2_passa(XL, XR, src_pad, dst_pad, att_flat, zeros16)
    denom = _k3_merge(dparts)
    XL2 = XL.reshape(N * H, C)
    S2all, Accall = _k4_passb(XL2, p, denom, src_pad, dst_pad, t16, zeros64)
    S2n = S2all.transpose(1, 2, 0, 3).reshape(N, HD)
    Accn = Accall.transpose(1, 2, 0, 3).reshape(N, HD)
    G = _k5_g(S2n, Accn, bias1, Wcat)
    gn = G[:, 1]
    nbrp = _k6_passc(gn, src_pad, dst_pad, zerosN)
    o = _k7_out(G, nbrp, bg, bout)
    k = 5000
    return o.reshape(_SORT_N)[:k].reshape(k, 1)
